# Initial kernel scaffold; baseline (speedup 1.0000x reference)
import functools

import jax
import jax.numpy as jnp
from jax import lax
from jax.experimental import pallas as pl
from jax.experimental.pallas import tpu as pltpu
from jax.experimental.pallas import tpu_sc as plsc

NC = 2
NS = 16
NW = NC * NS


def kernel(token_ids, weights):
    B0, H = token_ids.shape
    V, D = weights.shape
    B = B0 * H
    assert B % NW == 0
    b_per_w = B // NW
    CH = 16
    assert b_per_w % CH == 0
    R = b_per_w // CH

    mesh = plsc.VectorSubcoreMesh(
        core_axis_name="c", subcore_axis_name="s",
        num_cores=NC, num_subcores=NS)

    @functools.partial(
        pl.kernel,
        out_type=jax.ShapeDtypeStruct((NW, CH, R, D), jnp.float32),
        mesh=mesh,
        scratch_types=[
            pltpu.VMEM((CH, R), jnp.int32),
            pltpu.VMEM((R, D), jnp.float32),
            pltpu.VMEM((R, D), jnp.float32),
            pltpu.SemaphoreType.DMA,
            pltpu.SemaphoreType.DMA,
            pltpu.SemaphoreType.DMA,
            pltpu.SemaphoreType.DMA,
        ],
    )
    def emb(tids_hbm, table_hbm, out_hbm, idx_v, rows0, rows1,
            gs0, gs1, os0, os1):
        wid = lax.axis_index("s") * NC + lax.axis_index("c")
        pltpu.sync_copy(tids_hbm.at[wid], idx_v)

        bufs = (rows0, rows1)
        gsem = (gs0, gs1)
        osem = (os0, os1)

        def gstart(c):
            return pltpu.async_copy(
                table_hbm.at[idx_v.at[c]], bufs[c % 2], gsem[c % 2])

        def ostart(c):
            return pltpu.async_copy(
                bufs[c % 2], out_hbm.at[wid, c], osem[c % 2])

        g = {}
        o = {}
        g[0] = gstart(0)
        for c in range(CH):
            if c + 1 < CH:
                if c >= 1:
                    o[c - 1].wait()
                g[c + 1] = gstart(c + 1)
            g[c].wait()
            o[c] = ostart(c)
        o[CH - 2].wait()
        o[CH - 1].wait()

    tids = token_ids.reshape(NW, CH, R).astype(jnp.int32)
    out = emb(tids, weights)
    return out.reshape(B0, H, D)


# speedup vs baseline: 1.2547x; 1.2547x over previous
import functools

import jax
import jax.numpy as jnp
from jax import lax
from jax.experimental import pallas as pl
from jax.experimental.pallas import tpu as pltpu
from jax.experimental.pallas import tpu_sc as plsc

NC = 2
NS = 16
NW = NC * NS


def kernel(token_ids, weights):
    B0, H = token_ids.shape
    V, D = weights.shape
    B = B0 * H
    assert B % NW == 0
    b_per_w = B // NW
    CH = 16
    assert b_per_w % CH == 0
    R = b_per_w // CH

    mesh = plsc.VectorSubcoreMesh(
        core_axis_name="c", subcore_axis_name="s",
        num_cores=NC, num_subcores=NS)

    @functools.partial(
        pl.kernel,
        out_type=jax.ShapeDtypeStruct((NW, CH, R, D), jnp.float32),
        mesh=mesh,
        compiler_params=pltpu.CompilerParams(use_tc_tiling_on_sc=False),
        scratch_types=[
            pltpu.VMEM((CH, R), jnp.int32),
            pltpu.VMEM((R, D), jnp.float32),
            pltpu.VMEM((R, D), jnp.float32),
            pltpu.SemaphoreType.DMA,
            pltpu.SemaphoreType.DMA,
            pltpu.SemaphoreType.DMA,
            pltpu.SemaphoreType.DMA,
        ],
    )
    def emb(tids_hbm, table_hbm, out_hbm, idx_v, rows0, rows1,
            gs0, gs1, os0, os1):
        wid = lax.axis_index("s") * NC + lax.axis_index("c")
        pltpu.sync_copy(tids_hbm.at[wid], idx_v)

        bufs = (rows0, rows1)
        gsem = (gs0, gs1)
        osem = (os0, os1)

        def gstart(c):
            return pltpu.async_copy(
                table_hbm.at[idx_v.at[c]], bufs[c % 2], gsem[c % 2])

        def ostart(c):
            return pltpu.async_copy(
                bufs[c % 2], out_hbm.at[wid, c], osem[c % 2])

        g = {}
        o = {}
        g[0] = gstart(0)
        for c in range(CH):
            if c + 1 < CH:
                if c >= 1:
                    o[c - 1].wait()
                g[c + 1] = gstart(c + 1)
            g[c].wait()
            o[c] = ostart(c)
        o[CH - 2].wait()
        o[CH - 1].wait()

    tids = token_ids.reshape(NW, CH, R).astype(jnp.int32)
    out = emb(tids, weights)
    return out.reshape(B0, H, D)


# speedup vs baseline: 1.4634x; 1.1663x over previous
import functools

import jax
import jax.numpy as jnp
from jax import lax
from jax.experimental import pallas as pl
from jax.experimental.pallas import tpu as pltpu
from jax.experimental.pallas import tpu_sc as plsc

NC = 2
NS = 16
NW = NC * NS
L = 16


def kernel(token_ids, weights):
    B0, H = token_ids.shape
    V, D = weights.shape
    assert B0 % NW == 0
    W = B0 // NW

    mesh = plsc.VectorSubcoreMesh(
        core_axis_name="c", subcore_axis_name="s",
        num_cores=NC, num_subcores=NS)

    @functools.partial(
        pl.kernel,
        out_type=jax.ShapeDtypeStruct((H, D, B0), jnp.float32),
        mesh=mesh,
        compiler_params=pltpu.CompilerParams(
            use_tc_tiling_on_sc=False, needs_layout_passes=False),
        scratch_types=[
            pltpu.VMEM((H, W), jnp.int32),
            pltpu.VMEM((W, D), jnp.float32),
            pltpu.VMEM((W, D), jnp.float32),
            pltpu.VMEM((D, W), jnp.float32),
            pltpu.VMEM((D, W), jnp.float32),
            pltpu.SemaphoreType.DMA,
            pltpu.SemaphoreType.DMA,
            pltpu.SemaphoreType.DMA,
            pltpu.SemaphoreType.DMA,
            pltpu.SemaphoreType.DMA,
        ],
    )
    def emb(tids_hbm, table_hbm, out_hbm, idx_v, g0, g1, t0, t1,
            is_, gs0, gs1, os0, os1):
        wid = lax.axis_index("s") * NC + lax.axis_index("c")
        b0 = wid * W
        pltpu.async_copy(
            tids_hbm.at[:, pl.ds(b0, W)], idx_v, is_).wait()

        gbuf = (g0, g1)
        tbuf = (t0, t1)
        gsem = (gs0, gs1)
        osem = (os0, os1)

        def gstart(h):
            return pltpu.async_copy(
                table_hbm.at[idx_v.at[h]], gbuf[h % 2], gsem[h % 2])

        def ostart(h):
            return pltpu.async_copy(
                tbuf[h % 2], out_hbm.at[h, :, pl.ds(b0, W)], osem[h % 2])

        lane = lax.iota(jnp.int32, L)
        row_base = lane * D

        def transpose(g, t):
            def body_d(d, _):
                def body_k(k, _):
                    b_idx = k * L + lane
                    vec = plsc.load_gather(g, [b_idx, jnp.full((L,), d, jnp.int32)])
                    t[d, pl.ds(k * L, L)] = vec
                    return 0
                return lax.fori_loop(0, W // L, body_k, 0)
            lax.fori_loop(0, D, body_d, 0)

        g = {}
        o = {}
        g[0] = gstart(0)
        for h in range(H):
            g[h].wait()
            if h + 1 < H:
                g[h + 1] = gstart(h + 1)
            if h >= 2:
                o[h - 2].wait()
            transpose(gbuf[h % 2], tbuf[h % 2])
            o[h] = ostart(h)
        o[H - 2].wait()
        o[H - 1].wait()

    tids_t = jnp.swapaxes(token_ids, 0, 1).astype(jnp.int32)
    out = emb(tids_t, weights)
    return jnp.transpose(out, (2, 0, 1))


# speedup vs baseline: 1.6503x; 1.1277x over previous
import functools

import jax
import jax.numpy as jnp
from jax import lax
from jax.experimental import pallas as pl
from jax.experimental.pallas import tpu as pltpu
from jax.experimental.pallas import tpu_sc as plsc

NC = 2
NS = 16
NW = NC * NS
L = 16


def kernel(token_ids, weights):
    B0, H = token_ids.shape
    V, D = weights.shape
    assert B0 % NW == 0 and D == 32 and H % 2 == 0
    W = B0 // NW
    KB = W // L
    JT = W // 128

    mesh = plsc.VectorSubcoreMesh(
        core_axis_name="c", subcore_axis_name="s",
        num_cores=NC, num_subcores=NS)

    @functools.partial(
        pl.kernel,
        out_type=jax.ShapeDtypeStruct((H, D // 8, B0 // 128, 8, 128),
                                      jnp.float32),
        mesh=mesh,
        compiler_params=pltpu.CompilerParams(
            use_tc_tiling_on_sc=False, needs_layout_passes=False),
        scratch_types=[
            pltpu.VMEM((H, W), jnp.int32),
            pltpu.VMEM((W, D), jnp.float32),
            pltpu.VMEM((W, D), jnp.float32),
            pltpu.VMEM((D // 8, JT, 8, 128), jnp.float32),
            pltpu.VMEM((D // 8, JT, 8, 128), jnp.float32),
            pltpu.SemaphoreType.DMA,
            pltpu.SemaphoreType.DMA,
            pltpu.SemaphoreType.DMA,
            pltpu.SemaphoreType.DMA,
            pltpu.SemaphoreType.DMA,
        ],
    )
    def emb(tids_hbm, table_hbm, out_hbm, idx_v, g0, g1, t0, t1,
            is_, gs0, gs1, os0, os1):
        wid = lax.axis_index("s") * NC + lax.axis_index("c")
        b0 = wid * W
        jb = wid * JT
        pltpu.async_copy(tids_hbm.at[:, pl.ds(b0, W)], idx_v, is_).wait()

        gbuf = (g0, g1)
        tbuf = (t0, t1)
        gsem = (gs0, gs1)
        osem = (os0, os1)

        def gstart(h, p):
            return pltpu.async_copy(
                table_hbm.at[idx_v.at[h]], gbuf[p], gsem[p])

        def ostart(h, p):
            return pltpu.async_copy(
                tbuf[p], out_hbm.at[h, :, pl.ds(jb, JT)], osem[p])

        def gwait(h, p):
            pltpu.make_async_copy(
                table_hbm.at[idx_v.at[h]], gbuf[p], gsem[p]).wait()

        def owait(h, p):
            pltpu.make_async_copy(
                tbuf[p], out_hbm.at[h, :, pl.ds(jb, JT)], osem[p]).wait()

        lane = lax.iota(jnp.int32, L)

        def transpose(g, t):
            def body_d(d, _):
                dsplat = jnp.full((L,), d, jnp.int32)
                g8, r8 = d // 8, d % 8
                for k in range(KB):
                    bvec = lane + (k * L)
                    vec = plsc.load_gather(g, [bvec, dsplat])
                    t[g8, k // 8, r8, pl.ds((k % 8) * L, L)] = vec
                return 0
            lax.fori_loop(0, D, body_d, 0)

        gstart(0, 0)
        gstart(1, 1)

        def body_h2(h2, _):
            for p in range(2):
                h = h2 * 2 + p
                gwait(h, p)

                @pl.when(h2 > 0)
                def _():
                    owait(h - 2, p)

                transpose(gbuf[p], tbuf[p])
                ostart(h, p)

                @pl.when(h2 < (H // 2 - 1))
                def _():
                    gstart(h + 2, p)
            return 0

        lax.fori_loop(0, H // 2, body_h2, 0)
        owait(H - 2, 0)
        owait(H - 1, 1)

    tids_t = jnp.swapaxes(token_ids, 0, 1).astype(jnp.int32)
    out5 = emb(tids_t, weights)
    return jnp.transpose(out5, (2, 4, 0, 1, 3)).reshape(B0, H, D)


# speedup vs baseline: 2.2144x; 1.3418x over previous
import functools

import jax
import jax.numpy as jnp
from jax import lax
from jax.experimental import pallas as pl
from jax.experimental.pallas import tpu as pltpu
from jax.experimental.pallas import tpu_sc as plsc

NC = 2
NS = 16
NW = NC * NS
L = 16


def kernel(token_ids, weights):
    B0, H = token_ids.shape
    V, D = weights.shape
    assert B0 % NW == 0 and D == 32 and H % 2 == 0
    W = B0 // NW
    KB = W // L
    JT = W // 128

    mesh = plsc.VectorSubcoreMesh(
        core_axis_name="c", subcore_axis_name="s",
        num_cores=NC, num_subcores=NS)

    @functools.partial(
        pl.kernel,
        out_type=jax.ShapeDtypeStruct((H, D // 8, B0 // 128, 8, 128),
                                      jnp.float32),
        mesh=mesh,
        compiler_params=pltpu.CompilerParams(
            use_tc_tiling_on_sc=False, needs_layout_passes=False),
        scratch_types=[
            pltpu.VMEM((H, W), jnp.int32),
            pltpu.VMEM((W, D), jnp.float32),
            pltpu.VMEM((W, D), jnp.float32),
            pltpu.VMEM((D // 8, JT, 8, 128), jnp.float32),
            pltpu.VMEM((D // 8, JT, 8, 128), jnp.float32),
            pltpu.SemaphoreType.DMA,
            pltpu.SemaphoreType.DMA,
            pltpu.SemaphoreType.DMA,
            pltpu.SemaphoreType.DMA,
            pltpu.SemaphoreType.DMA,
        ],
    )
    def emb(tids_hbm, table_hbm, out_hbm, idx_v, g0, g1, t0, t1,
            is_, gs0, gs1, os0, os1):
        wid = lax.axis_index("s") * NC + lax.axis_index("c")
        b0 = wid * W
        jb = wid * JT
        pltpu.async_copy(tids_hbm.at[:, pl.ds(b0, W)], idx_v, is_).wait()

        gbuf = (g0, g1)
        tbuf = (t0, t1)
        gsem = (gs0, gs1)
        osem = (os0, os1)

        def gstart(h, p):
            return pltpu.async_copy(
                table_hbm.at[idx_v.at[h]], gbuf[p], gsem[p])

        def ostart(h, p):
            return pltpu.async_copy(
                tbuf[p], out_hbm.at[h, :, pl.ds(jb, JT)], osem[p])

        def gwait(h, p):
            pltpu.make_async_copy(
                table_hbm.at[idx_v.at[h]], gbuf[p], gsem[p]).wait()

        def owait(h, p):
            pltpu.make_async_copy(
                tbuf[p], out_hbm.at[h, :, pl.ds(jb, JT)], osem[p]).wait()

        lane = lax.iota(jnp.int32, L)

        def transpose(g, t):
            @plsc.parallel_loop(0, D, 1, unroll=2)
            def body_d(d):
                dsplat = jnp.full((L,), d, jnp.int32)
                g8, r8 = d // 8, d % 8
                for k in range(KB):
                    bvec = lane + (k * L)
                    vec = plsc.load_gather(g, [bvec, dsplat])
                    t[g8, k // 8, r8, pl.ds((k % 8) * L, L)] = vec

        gstart(0, 0)
        gstart(1, 1)

        def body_h2(h2, _):
            for p in range(2):
                h = h2 * 2 + p
                gwait(h, p)

                @pl.when(h2 > 0)
                def _():
                    owait(h - 2, p)

                transpose(gbuf[p], tbuf[p])
                ostart(h, p)

                @pl.when(h2 < (H // 2 - 1))
                def _():
                    gstart(h + 2, p)
            return 0

        lax.fori_loop(0, H // 2, body_h2, 0)
        owait(H - 2, 0)
        owait(H - 1, 1)

    tids_t = jnp.swapaxes(token_ids, 0, 1).astype(jnp.int32)
    out5 = emb(tids_t, weights)
    return jnp.transpose(out5, (2, 4, 0, 1, 3)).reshape(B0, H, D)


# speedup vs baseline: 2.4594x; 1.1106x over previous
import functools

import jax
import jax.numpy as jnp
from jax import lax
from jax.experimental import pallas as pl
from jax.experimental.pallas import tpu as pltpu
from jax.experimental.pallas import tpu_sc as plsc

NC = 2
NS = 16
NW = NC * NS
L = 16


def _detile_table(weights_t):
    D, V = weights_t.shape
    assert D == 32
    CW = 128
    NCOLS = V // CW
    TAIL = V - NCOLS * CW
    PER_W = (NCOLS + NW - 1) // NW

    mesh = plsc.VectorSubcoreMesh(
        core_axis_name="c", subcore_axis_name="s",
        num_cores=NC, num_subcores=NS)

    @functools.partial(
        pl.kernel,
        out_type=jax.ShapeDtypeStruct((V * D,), jnp.float32),
        mesh=mesh,
        compiler_params=pltpu.CompilerParams(
            use_tc_tiling_on_sc=True, needs_layout_passes=False),
        scratch_types=[
            pltpu.VMEM((D, CW), jnp.float32),
            pltpu.VMEM((D, CW), jnp.float32),
            pltpu.VMEM((CW * D,), jnp.float32),
            pltpu.VMEM((CW * D,), jnp.float32),
            pltpu.SemaphoreType.DMA,
            pltpu.SemaphoreType.DMA,
            pltpu.SemaphoreType.DMA,
            pltpu.SemaphoreType.DMA,
        ],
    )
    def detile(wt_hbm, out_hbm, g0, g1, t0, t1, gs0, gs1, os0, os1):
        wid = lax.axis_index("s") * NC + lax.axis_index("c")
        c_lo = wid * PER_W
        c_hi = jnp.minimum(c_lo + PER_W, NCOLS)

        gbuf = (g0, g1)
        tbuf = (t0, t1)
        gsem = (gs0, gs1)
        osem = (os0, os1)
        lane = lax.iota(jnp.int32, L)

        def gcopy(c, p):
            return pltpu.make_async_copy(
                wt_hbm.at[:, pl.ds(c * CW, CW)], gbuf[p], gsem[p])

        def ocopy(c, p):
            return pltpu.make_async_copy(
                tbuf[p], out_hbm.at[pl.ds(c * (CW * D), CW * D)], osem[p])

        def transpose(g, t):
            @plsc.parallel_loop(0, CW, 1, unroll=2)
            def body_v(v):
                vsplat = jnp.full((L,), v, jnp.int32)
                for half in range(2):
                    dvec = lane + (half * L)
                    vec = plsc.load_gather(g, [dvec, vsplat])
                    t[pl.ds(v * D + half * L, L)] = vec

        n = c_hi - c_lo

        @pl.when(n > 0)
        def _():
            gcopy(c_lo, 0).start()

        def body_c(i, _):
            c = c_lo + i
            for p in range(2):
                @pl.when(lax.rem(i, 2) == p)
                def _():
                    gcopy(c, p).wait()

                    @pl.when(i + 1 < n)
                    def _():
                        gcopy(c + 1, 1 - p).start()

                    @pl.when(i >= 2)
                    def _():
                        ocopy(c - 2, p).wait()
                    transpose(gbuf[p], tbuf[p])
                    ocopy(c, p).start()
            return 0

        lax.fori_loop(0, n, body_c, 0)

        for q in range(2):
            @pl.when((n >= 2) & (lax.rem(n - 2, 2) == q))
            def _(q=q):
                ocopy(c_hi - 2, q).wait()

            @pl.when((n >= 1) & (lax.rem(n - 1, 2) == q))
            def _(q=q):
                ocopy(c_hi - 1, q).wait()

    return detile(weights_t)


def kernel(token_ids, weights):
    B0, H = token_ids.shape
    V, D = weights.shape
    assert B0 % NW == 0 and D == 32 and H % 2 == 0
    W = B0 // NW
    KB = W // L
    JT = W // 128

    mesh = plsc.VectorSubcoreMesh(
        core_axis_name="c", subcore_axis_name="s",
        num_cores=NC, num_subcores=NS)

    @functools.partial(
        pl.kernel,
        out_type=jax.ShapeDtypeStruct((H, D // 8, B0 // 128, 8, 128),
                                      jnp.float32),
        mesh=mesh,
        compiler_params=pltpu.CompilerParams(
            use_tc_tiling_on_sc=False, needs_layout_passes=False),
        scratch_types=[
            pltpu.VMEM((H, W), jnp.int32),
            pltpu.VMEM((W, D), jnp.float32),
            pltpu.VMEM((W, D), jnp.float32),
            pltpu.VMEM((D // 8, JT, 8, 128), jnp.float32),
            pltpu.VMEM((D // 8, JT, 8, 128), jnp.float32),
            pltpu.SemaphoreType.DMA,
            pltpu.SemaphoreType.DMA,
            pltpu.SemaphoreType.DMA,
            pltpu.SemaphoreType.DMA,
            pltpu.SemaphoreType.DMA,
        ],
    )
    def emb(tids_hbm, table_hbm, out_hbm, idx_v, g0, g1, t0, t1,
            is_, gs0, gs1, os0, os1):
        wid = lax.axis_index("s") * NC + lax.axis_index("c")
        b0 = wid * W
        jb = wid * JT
        pltpu.async_copy(tids_hbm.at[:, pl.ds(b0, W)], idx_v, is_).wait()

        gbuf = (g0, g1)
        tbuf = (t0, t1)
        gsem = (gs0, gs1)
        osem = (os0, os1)

        def gstart(h, p):
            return pltpu.async_copy(
                table_hbm.at[idx_v.at[h]], gbuf[p], gsem[p])

        def ostart(h, p):
            return pltpu.async_copy(
                tbuf[p], out_hbm.at[h, :, pl.ds(jb, JT)], osem[p])

        def gwait(h, p):
            pltpu.make_async_copy(
                table_hbm.at[idx_v.at[h]], gbuf[p], gsem[p]).wait()

        def owait(h, p):
            pltpu.make_async_copy(
                tbuf[p], out_hbm.at[h, :, pl.ds(jb, JT)], osem[p]).wait()

        lane = lax.iota(jnp.int32, L)

        def transpose(g, t):
            @plsc.parallel_loop(0, D, 1, unroll=2)
            def body_d(d):
                dsplat = jnp.full((L,), d, jnp.int32)
                g8, r8 = d // 8, d % 8
                for k in range(KB):
                    bvec = lane + (k * L)
                    vec = plsc.load_gather(g, [bvec, dsplat])
                    t[g8, k // 8, r8, pl.ds((k % 8) * L, L)] = vec

        gstart(0, 0)
        gstart(1, 1)

        def body_h2(h2, _):
            for p in range(2):
                h = h2 * 2 + p
                gwait(h, p)

                @pl.when(h2 > 0)
                def _():
                    owait(h - 2, p)

                transpose(gbuf[p], tbuf[p])
                ostart(h, p)

                @pl.when(h2 < (H // 2 - 1))
                def _():
                    gstart(h + 2, p)
            return 0

        lax.fori_loop(0, H // 2, body_h2, 0)
        owait(H - 2, 0)
        owait(H - 1, 1)

    tids_t = jnp.swapaxes(token_ids, 0, 1).astype(jnp.int32)
    w_flat = _detile_table(jnp.swapaxes(weights, 0, 1))
    ntail = V % 128
    if ntail:
        tail = weights[V - ntail:, :].reshape(-1)
        w_flat = jax.lax.dynamic_update_slice(w_flat, tail, ((V - ntail) * D,))
    w_lin = w_flat.reshape(V, D)
    out5 = emb(tids_t, w_lin)
    return jnp.transpose(out5, (2, 4, 0, 1, 3)).reshape(B0, H, D)
